# baseline (speedup 1.0000x reference)
import functools

import numpy as np
import jax
import jax.numpy as jnp
from jax import lax
from jax.experimental import pallas as pl
from jax.experimental.pallas import tpu as pltpu
from jax.experimental.pallas import tpu_sc as plsc

NUM_ADDRESSES = 100000
ADDRESS_DIM = 512
CONTENT_DIM = 512
BATCH = 256
THRESHOLD = 76

BLK = 4000
WPB = 128
NBLK = NUM_ADDRESSES // BLK
WORDS_PER_ROW = NBLK * WPB
SUMS_PER_BLK = WPB // 32
NSUM = NBLK * SUMS_PER_BLK
SUMS_PER_ROW = 104
NW = 32
ROWS_PER_TILE = BATCH // NW
TILE_WORDS = ROWS_PER_TILE * WORDS_PER_ROW

CAPW = 1024
CAPP = 1024
CAP = 96


def _pack_weights():
    c = np.arange(BLK)
    g = c // 32
    bit = c % 32
    glo = np.zeros((BLK, WPB), np.float32)
    ghi = np.zeros((BLK, WPB), np.float32)
    lo = bit < 16
    glo[c[lo], g[lo]] = (2.0 ** bit[lo])
    ghi[c[~lo], g[~lo]] = (2.0 ** (bit[~lo] - 16))
    return glo, ghi


def _summary_weights():
    w = np.arange(WORDS_PER_ROW)
    s = w // 32
    bit = w % 32
    plo = np.zeros((WORDS_PER_ROW, SUMS_PER_ROW), np.float32)
    phi = np.zeros((WORDS_PER_ROW, SUMS_PER_ROW), np.float32)
    lo = bit < 16
    plo[w[lo], s[lo]] = (2.0 ** bit[lo])
    phi[w[~lo], s[~lo]] = (2.0 ** (bit[~lo] - 16))
    return plo, phi


_GLO_NP, _GHI_NP = _pack_weights()
_PLO_NP, _PHI_NP = _summary_weights()


def _tc_pack_body(address_ref, addresses_ref, glo_ref, ghi_ref, out_ref):
    sim = lax.dot_general(
        address_ref[...].astype(jnp.bfloat16),
        addresses_ref[...].astype(jnp.bfloat16),
        (((1,), (1,)), ((), ())),
        preferred_element_type=jnp.float32,
    )
    mask = (sim >= THRESHOLD).astype(jnp.bfloat16)
    lo = lax.dot_general(mask, glo_ref[...], (((1,), (0,)), ((), ())),
                         preferred_element_type=jnp.float32)
    hi = lax.dot_general(mask, ghi_ref[...], (((1,), (0,)), ((), ())),
                         preferred_element_type=jnp.float32)
    out_ref[...] = lo.astype(jnp.int32) | (hi.astype(jnp.int32) << 16)


def _tc_pack(address, addresses, glo, ghi):
    return pl.pallas_call(
        _tc_pack_body,
        grid=(NBLK,),
        in_specs=[
            pl.BlockSpec((BATCH, ADDRESS_DIM), lambda j: (0, 0)),
            pl.BlockSpec((BLK, ADDRESS_DIM), lambda j: (j, 0)),
            pl.BlockSpec((BLK, WPB), lambda j: (0, 0)),
            pl.BlockSpec((BLK, WPB), lambda j: (0, 0)),
        ],
        out_specs=pl.BlockSpec((BATCH, WPB), lambda j: (0, j)),
        out_shape=jax.ShapeDtypeStruct((BATCH, WORDS_PER_ROW), jnp.int32),
    )(address, addresses, glo, ghi)


def _tc_l2_body(l1_ref, plo_ref, phi_ref, l2_ref):
    ind = (l1_ref[...] != 0).astype(jnp.bfloat16)
    l2lo = lax.dot_general(ind, plo_ref[...], (((1,), (0,)), ((), ())),
                           preferred_element_type=jnp.float32)
    l2hi = lax.dot_general(ind, phi_ref[...], (((1,), (0,)), ((), ())),
                           preferred_element_type=jnp.float32)
    l2_ref[...] = l2lo.astype(jnp.int32) | (l2hi.astype(jnp.int32) << 16)


def _tc_l2(l1, plo, phi):
    return pl.pallas_call(
        _tc_l2_body,
        in_specs=[
            pl.BlockSpec((BATCH, WORDS_PER_ROW), lambda: (0, 0)),
            pl.BlockSpec((WORDS_PER_ROW, SUMS_PER_ROW), lambda: (0, 0)),
            pl.BlockSpec((WORDS_PER_ROW, SUMS_PER_ROW), lambda: (0, 0)),
        ],
        out_specs=pl.BlockSpec((BATCH, SUMS_PER_ROW), lambda: (0, 0)),
        out_shape=jax.ShapeDtypeStruct((BATCH, SUMS_PER_ROW), jnp.int32),
    )(l1, plo, phi)


def _sc_body(l1_hbm, l2_hbm, content_hbm, out_hbm,
             maskbuf, l2buf, sbuf, wbuf, colbuf, idxbuf, gbuf, accbuf, *sems):
    wid = lax.axis_index("s") * 2 + lax.axis_index("c")
    row0 = wid * ROWS_PER_TILE
    lanes = lax.iota(jnp.int32, 16)
    zeros16 = jnp.zeros((16,), jnp.int32)
    ones16 = jnp.ones((16,), jnp.int32)
    fzeros = jnp.zeros((16,), jnp.float32)
    fones = jnp.ones((16,), jnp.float32)

    pltpu.sync_copy(l1_hbm.at[pl.ds(wid * TILE_WORDS, TILE_WORDS)], maskbuf)

    def row_body(r, carry):
        row = row0 + r
        rowbase = r * WORDS_PER_ROW

        l2buf[pl.ds(96, 16)] = zeros16
        pltpu.sync_copy(l2_hbm.at[pl.ds(row * SUMS_PER_ROW, SUMS_PER_ROW)],
                        l2buf.at[pl.ds(0, SUMS_PER_ROW)])

        def s_scan(i, scnt):
            v = l2buf[pl.ds(i * 16, 16)]
            nz = v != 0
            nzi = jnp.where(nz, ones16, zeros16)
            cnt = jnp.sum(nzi)

            @pl.when(cnt > 0)
            def _():
                pos = scnt + plsc.cumsum(nzi) - nzi
                plsc.store_scatter(sbuf, [pos], i * 16 + lanes, mask=nz)

            return scnt + cnt

        scnt = lax.fori_loop(0, 7, s_scan, 0)
        plsc.subcore_barrier()

        def sdec(j, wcnt):
            raw = sbuf[pl.ds(j * 16, 16)]
            in_range = (j * 16 + lanes) < scnt
            sv = jnp.where(in_range, raw, 111)
            vals = plsc.load_gather(l2buf, [sv])
            vals = jnp.where(in_range, vals, zeros16)
            wbase = 32 * sv

            t = vals - ((vals >> 1) & 0x55555555)
            t = (t & 0x33333333) + ((t >> 2) & 0x33333333)
            t = (t + (t >> 4)) & 0x0F0F0F0F
            nbits = (t * 0x01010101) >> 24

            offs = wcnt + plsc.cumsum(nbits) - nbits
            guard = wcnt <= CAPW - 512

            @pl.when(guard)
            def _():
                rc = offs
                for b in range(32):
                    bit = (vals >> b) & 1
                    plsc.store_scatter(wbuf, [rc], wbase + b, mask=bit == 1)
                    rc = rc + bit

            return wcnt + jnp.where(guard, jnp.sum(nbits), 0)

        wcnt = lax.fori_loop(0, (scnt + 15) // 16, sdec, 0)
        plsc.subcore_barrier()

        def dec_body(j, pcnt):
            raw = wbuf[pl.ds(j * 16, 16)]
            in_range = (j * 16 + lanes) < wcnt
            wv = jnp.where(in_range, raw, WORDS_PER_ROW - 1)
            vals = plsc.load_gather(maskbuf, [rowbase + wv])
            vals = jnp.where(in_range, vals, zeros16)
            colbase = BLK * (wv >> 7) + 32 * (wv & 127)

            t = vals - ((vals >> 1) & 0x55555555)
            t = (t & 0x33333333) + ((t >> 2) & 0x33333333)
            t = (t + (t >> 4)) & 0x0F0F0F0F
            nbits = (t * 0x01010101) >> 24

            offs = pcnt + plsc.cumsum(nbits) - nbits
            guard = pcnt <= CAPP - 512

            @pl.when(guard)
            def _():
                rc = offs
                for b in range(32):
                    bit = (vals >> b) & 1
                    plsc.store_scatter(colbuf, [rc], colbase + b,
                                       mask=bit == 1)
                    rc = rc + bit

            return pcnt + jnp.where(guard, jnp.sum(nbits), 0)

        pcnt = lax.fori_loop(0, (wcnt + 15) // 16, dec_body, 0)
        plsc.subcore_barrier()

        for q in range(CAP // 16):
            live = (q * 16 + lanes) < pcnt
            cc = plsc.load_gather(colbuf, [q * 16 + lanes])
            idxbuf[q, pl.ds(0, 16)] = jnp.where(live, cc, zeros16)

        handles = [
            pltpu.async_copy(content_hbm.at[idxbuf.at[c]],
                             gbuf.at[pl.ds(c * 16, 16)], sems[c])
            for c in range(CAP // 16)
        ]
        for h in handles:
            h.wait()

        bound = jnp.minimum(pcnt, CAP)
        for v in range(CONTENT_DIM // 16):
            accbuf[pl.ds(v * 16, 16)] = fzeros

        def acc_step(j, carry2):
            for v in range(CONTENT_DIM // 16):
                g = gbuf[j, pl.ds(v * 16, 16)]
                bits = lax.bitcast_convert_type(g, jnp.int32)
                bits = bits + 0x7FFF + ((bits >> 16) & 1)
                bits = bits & jnp.int32(-65536)
                accbuf[pl.ds(v * 16, 16)] = (
                    accbuf[pl.ds(v * 16, 16)]
                    + lax.bitcast_convert_type(bits, jnp.float32))
            return carry2

        lax.fori_loop(0, bound, acc_step, 0)

        for v in range(CONTENT_DIM // 16):
            acc = accbuf[pl.ds(v * 16, 16)]
            accbuf[pl.ds(v * 16, 16)] = jnp.where(
                acc > 0.0, fones, jnp.where(acc < 0.0, -fones, fzeros))

        pltpu.sync_copy(accbuf, out_hbm.at[row])
        return carry

    lax.fori_loop(0, ROWS_PER_TILE, row_body, 0)


@functools.lru_cache(maxsize=1)
def _make_sc_read():
    @functools.partial(
        pl.kernel,
        out_type=jax.ShapeDtypeStruct((BATCH, CONTENT_DIM), jnp.float32),
        mesh=plsc.VectorSubcoreMesh(core_axis_name="c", subcore_axis_name="s"),
        compiler_params=pltpu.CompilerParams(needs_layout_passes=False),
        scratch_types=[
            pltpu.VMEM((TILE_WORDS,), jnp.int32),
            pltpu.VMEM((112,), jnp.int32),
            pltpu.VMEM((128,), jnp.int32),
            pltpu.VMEM((CAPW,), jnp.int32),
            pltpu.VMEM((CAPP,), jnp.int32),
            pltpu.VMEM((CAP // 16, 16), jnp.int32),
            pltpu.VMEM((CAP, CONTENT_DIM), jnp.float32),
            pltpu.VMEM((CONTENT_DIM,), jnp.float32),
        ] + [pltpu.SemaphoreType.DMA] * (CAP // 16),
    )
    def _sc_read(l1_flat, l2_flat, content, out, *scratch):
        _sc_body(l1_flat, l2_flat, content, out, *scratch)

    return _sc_read


@jax.jit
def kernel(address, addresses, content):
    glo = jnp.asarray(_GLO_NP).astype(jnp.bfloat16)
    ghi = jnp.asarray(_GHI_NP).astype(jnp.bfloat16)
    plo = jnp.asarray(_PLO_NP).astype(jnp.bfloat16)
    phi = jnp.asarray(_PHI_NP).astype(jnp.bfloat16)
    l1 = _tc_pack(address, addresses, glo, ghi)
    l2 = _tc_l2(l1, plo, phi)
    return _make_sc_read()(l1.reshape(-1), l2.reshape(-1), content)


# speedup vs baseline: 1.1457x; 1.1457x over previous
import functools

import numpy as np
import jax
import jax.numpy as jnp
from jax import lax
from jax.experimental import pallas as pl
from jax.experimental.pallas import tpu as pltpu
from jax.experimental.pallas import tpu_sc as plsc

NUM_ADDRESSES = 100000
ADDRESS_DIM = 512
CONTENT_DIM = 512
BATCH = 256
THRESHOLD = 76

BLK = 4000
WPB = 128
NBLK = NUM_ADDRESSES // BLK
WORDS_PER_ROW = NBLK * WPB
SUMS_PER_BLK = WPB // 32
NSUM = NBLK * SUMS_PER_BLK
SUMS_PER_ROW = 104
NW = 32
ROWS_PER_TILE = BATCH // NW
TILE_WORDS = ROWS_PER_TILE * WORDS_PER_ROW

CAPW = 1024
CAPP = 1024
CAP = 96


def _pack_weights():
    c = np.arange(BLK)
    g = c // 32
    bit = c % 32
    glo = np.zeros((BLK, WPB), np.float32)
    ghi = np.zeros((BLK, WPB), np.float32)
    lo = bit < 16
    glo[c[lo], g[lo]] = (2.0 ** bit[lo])
    ghi[c[~lo], g[~lo]] = (2.0 ** (bit[~lo] - 16))
    return glo, ghi


def _summary_weights():
    w = np.arange(WORDS_PER_ROW)
    s = w // 32
    bit = w % 32
    plo = np.zeros((WORDS_PER_ROW, SUMS_PER_ROW), np.float32)
    phi = np.zeros((WORDS_PER_ROW, SUMS_PER_ROW), np.float32)
    lo = bit < 16
    plo[w[lo], s[lo]] = (2.0 ** bit[lo])
    phi[w[~lo], s[~lo]] = (2.0 ** (bit[~lo] - 16))
    return plo, phi


_GLO_NP, _GHI_NP = _pack_weights()
_PLO_NP, _PHI_NP = _summary_weights()


def _tc_pack_body(address_ref, addresses_ref, glo_ref, ghi_ref, out_ref):
    sim = lax.dot_general(
        address_ref[...].astype(jnp.bfloat16),
        addresses_ref[...].astype(jnp.bfloat16),
        (((1,), (1,)), ((), ())),
        preferred_element_type=jnp.float32,
    )
    mask = (sim >= THRESHOLD).astype(jnp.bfloat16)
    lo = lax.dot_general(mask, glo_ref[...], (((1,), (0,)), ((), ())),
                         preferred_element_type=jnp.float32)
    hi = lax.dot_general(mask, ghi_ref[...], (((1,), (0,)), ((), ())),
                         preferred_element_type=jnp.float32)
    out_ref[...] = lo.astype(jnp.int32) | (hi.astype(jnp.int32) << 16)


def _tc_pack(address, addresses, glo, ghi):
    return pl.pallas_call(
        _tc_pack_body,
        grid=(NBLK,),
        in_specs=[
            pl.BlockSpec((BATCH, ADDRESS_DIM), lambda j: (0, 0)),
            pl.BlockSpec((BLK, ADDRESS_DIM), lambda j: (j, 0)),
            pl.BlockSpec((BLK, WPB), lambda j: (0, 0)),
            pl.BlockSpec((BLK, WPB), lambda j: (0, 0)),
        ],
        out_specs=pl.BlockSpec((BATCH, WPB), lambda j: (0, j)),
        out_shape=jax.ShapeDtypeStruct((BATCH, WORDS_PER_ROW), jnp.int32),
    )(address, addresses, glo, ghi)


def _tc_l2_body(l1_ref, plo_ref, phi_ref, l2_ref):
    ind = (l1_ref[...] != 0).astype(jnp.bfloat16)
    l2lo = lax.dot_general(ind, plo_ref[...], (((1,), (0,)), ((), ())),
                           preferred_element_type=jnp.float32)
    l2hi = lax.dot_general(ind, phi_ref[...], (((1,), (0,)), ((), ())),
                           preferred_element_type=jnp.float32)
    l2_ref[...] = l2lo.astype(jnp.int32) | (l2hi.astype(jnp.int32) << 16)


def _tc_l2(l1, plo, phi):
    return pl.pallas_call(
        _tc_l2_body,
        in_specs=[
            pl.BlockSpec((BATCH, WORDS_PER_ROW), lambda: (0, 0)),
            pl.BlockSpec((WORDS_PER_ROW, SUMS_PER_ROW), lambda: (0, 0)),
            pl.BlockSpec((WORDS_PER_ROW, SUMS_PER_ROW), lambda: (0, 0)),
        ],
        out_specs=pl.BlockSpec((BATCH, SUMS_PER_ROW), lambda: (0, 0)),
        out_shape=jax.ShapeDtypeStruct((BATCH, SUMS_PER_ROW), jnp.int32),
    )(l1, plo, phi)


def _sc_body(l1_hbm, l2_hbm, content_hbm, out_hbm,
             maskbuf, l2buf, sbuf, wbuf, colbuf, idxbuf, gbuf, accbuf, *sems):
    wid = lax.axis_index("s") * 2 + lax.axis_index("c")
    row0 = wid * ROWS_PER_TILE
    lanes = lax.iota(jnp.int32, 16)
    zeros16 = jnp.zeros((16,), jnp.int32)
    ones16 = jnp.ones((16,), jnp.int32)
    fzeros = jnp.zeros((16,), jnp.float32)
    fones = jnp.ones((16,), jnp.float32)

    pltpu.sync_copy(l1_hbm.at[pl.ds(wid * TILE_WORDS, TILE_WORDS)], maskbuf)

    def row_body(r, carry):
        row = row0 + r
        rowbase = r * WORDS_PER_ROW

        l2buf[pl.ds(96, 16)] = zeros16
        pltpu.sync_copy(l2_hbm.at[pl.ds(row * SUMS_PER_ROW, SUMS_PER_ROW)],
                        l2buf.at[pl.ds(0, SUMS_PER_ROW)])

        def s_scan(i, scnt):
            v = l2buf[pl.ds(i * 16, 16)]
            nz = v != 0
            nzi = jnp.where(nz, ones16, zeros16)
            cnt = jnp.sum(nzi)

            @pl.when(cnt > 0)
            def _():
                pos = scnt + plsc.cumsum(nzi) - nzi
                plsc.store_scatter(sbuf, [pos], i * 16 + lanes, mask=nz)

            return scnt + cnt

        scnt = lax.fori_loop(0, 7, s_scan, 0)

        def sdec(j, wcnt):
            raw = sbuf[pl.ds(j * 16, 16)]
            in_range = (j * 16 + lanes) < scnt
            sv = jnp.where(in_range, raw, 111)
            vals = plsc.load_gather(l2buf, [sv])
            vals = jnp.where(in_range, vals, zeros16)
            wbase = 32 * sv

            t = vals - ((vals >> 1) & 0x55555555)
            t = (t & 0x33333333) + ((t >> 2) & 0x33333333)
            t = (t + (t >> 4)) & 0x0F0F0F0F
            nbits = (t * 0x01010101) >> 24

            offs = wcnt + plsc.cumsum(nbits) - nbits
            guard = wcnt <= CAPW - 512

            @pl.when(guard)
            def _():
                rc = offs
                for b in range(32):
                    bit = (vals >> b) & 1
                    plsc.store_scatter(wbuf, [rc], wbase + b, mask=bit == 1)
                    rc = rc + bit

            return wcnt + jnp.where(guard, jnp.sum(nbits), 0)

        wcnt = lax.fori_loop(0, (scnt + 15) // 16, sdec, 0)

        def dec_body(j, pcnt):
            raw = wbuf[pl.ds(j * 16, 16)]
            in_range = (j * 16 + lanes) < wcnt
            wv = jnp.where(in_range, raw, WORDS_PER_ROW - 1)
            vals = plsc.load_gather(maskbuf, [rowbase + wv])
            vals = jnp.where(in_range, vals, zeros16)
            colbase = BLK * (wv >> 7) + 32 * (wv & 127)

            t = vals - ((vals >> 1) & 0x55555555)
            t = (t & 0x33333333) + ((t >> 2) & 0x33333333)
            t = (t + (t >> 4)) & 0x0F0F0F0F
            nbits = (t * 0x01010101) >> 24

            offs = pcnt + plsc.cumsum(nbits) - nbits
            guard = pcnt <= CAPP - 512

            @pl.when(guard)
            def _():
                rc = offs
                for b in range(32):
                    bit = (vals >> b) & 1
                    plsc.store_scatter(colbuf, [rc], colbase + b,
                                       mask=bit == 1)
                    rc = rc + bit

            return pcnt + jnp.where(guard, jnp.sum(nbits), 0)

        pcnt = lax.fori_loop(0, (wcnt + 15) // 16, dec_body, 0)

        for q in range(CAP // 16):
            live = (q * 16 + lanes) < pcnt
            cc = plsc.load_gather(colbuf, [q * 16 + lanes])
            idxbuf[q, pl.ds(0, 16)] = jnp.where(live, cc, zeros16)

        handles = [
            pltpu.async_copy(content_hbm.at[idxbuf.at[c]],
                             gbuf.at[pl.ds(c * 16, 16)], sems[c])
            for c in range(CAP // 16)
        ]
        for h in handles:
            h.wait()

        bound = jnp.minimum(pcnt, CAP)
        for v in range(CONTENT_DIM // 16):
            def acc_step(j, acc, v=v):
                g = gbuf[j, pl.ds(v * 16, 16)]
                bits = lax.bitcast_convert_type(g, jnp.int32)
                bits = bits + 0x7FFF + ((bits >> 16) & 1)
                bits = bits & jnp.int32(-65536)
                return acc + lax.bitcast_convert_type(bits, jnp.float32)

            acc = lax.fori_loop(0, bound, acc_step, fzeros)
            accbuf[pl.ds(v * 16, 16)] = jnp.where(
                acc > 0.0, fones, jnp.where(acc < 0.0, -fones, fzeros))

        pltpu.sync_copy(accbuf, out_hbm.at[row])
        return carry

    lax.fori_loop(0, ROWS_PER_TILE, row_body, 0)


@functools.lru_cache(maxsize=1)
def _make_sc_read():
    @functools.partial(
        pl.kernel,
        out_type=jax.ShapeDtypeStruct((BATCH, CONTENT_DIM), jnp.float32),
        mesh=plsc.VectorSubcoreMesh(core_axis_name="c", subcore_axis_name="s"),
        compiler_params=pltpu.CompilerParams(needs_layout_passes=False),
        scratch_types=[
            pltpu.VMEM((TILE_WORDS,), jnp.int32),
            pltpu.VMEM((112,), jnp.int32),
            pltpu.VMEM((128,), jnp.int32),
            pltpu.VMEM((CAPW,), jnp.int32),
            pltpu.VMEM((CAPP,), jnp.int32),
            pltpu.VMEM((CAP // 16, 16), jnp.int32),
            pltpu.VMEM((CAP, CONTENT_DIM), jnp.float32),
            pltpu.VMEM((CONTENT_DIM,), jnp.float32),
        ] + [pltpu.SemaphoreType.DMA] * (CAP // 16),
    )
    def _sc_read(l1_flat, l2_flat, content, out, *scratch):
        _sc_body(l1_flat, l2_flat, content, out, *scratch)

    return _sc_read


@jax.jit
def kernel(address, addresses, content):
    glo = jnp.asarray(_GLO_NP).astype(jnp.bfloat16)
    ghi = jnp.asarray(_GHI_NP).astype(jnp.bfloat16)
    plo = jnp.asarray(_PLO_NP).astype(jnp.bfloat16)
    phi = jnp.asarray(_PHI_NP).astype(jnp.bfloat16)
    l1 = _tc_pack(address, addresses, glo, ghi)
    l2 = _tc_l2(l1, plo, phi)
    return _make_sc_read()(l1.reshape(-1), l2.reshape(-1), content)
